# Initial kernel scaffold; baseline (speedup 1.0000x reference)
import functools

import jax
import jax.numpy as jnp
from jax import lax
from jax.experimental import pallas as pl
from jax.experimental.pallas import tpu as pltpu
from jax.experimental.pallas import tpu_sc as plsc

N = 10000
D = 128
H = 4
C = 32
MLPD = 512

NC = 2
NS = 16
NW = NC * NS

N_PAD = 10240
ROWW = 144
CHUNK = 128
BLK = 1024

_NEG = -1e30


def _k1_body(x_ref, w_ref, as_ref, ad_ref, t1_ref, t2_ref):
    i = pl.program_id(0)
    h = jnp.dot(x_ref[...], w_ref[...], preferred_element_type=jnp.float32,
                precision=lax.Precision.HIGHEST)
    a_s = jnp.dot(h, as_ref[...], preferred_element_type=jnp.float32,
                  precision=lax.Precision.HIGHEST)
    a_d = jnp.dot(h, ad_ref[...], preferred_element_type=jnp.float32,
                  precision=lax.Precision.HIGHEST)
    rows = i * BLK + lax.broadcasted_iota(jnp.int32, (BLK, 1), 0)
    sent = rows >= N
    a_s = jnp.where(sent, _NEG, a_s)
    a_d = jnp.where(sent, _NEG, a_d)
    z12 = jnp.zeros((BLK, 12), jnp.float32)
    t1_ref[...] = jnp.concatenate([h, a_s, z12], axis=1)
    t2_ref[...] = jnp.concatenate([a_d, z12], axis=1)


def _k2_body(t1_hbm, t2_hbm, src_hbm, dst_hbm, out_hbm,
             acc, hbuf, adbuf, rowbuf, zbuf, sidx, didx, sem,
             *, n_chunks_per_tile):
    c = lax.axis_index("c")
    s = lax.axis_index("s")
    z16 = jnp.zeros((16,), jnp.float32)

    def zrow(r, _):
        for j in range(ROWW // 16):
            zbuf[r, pl.ds(16 * j, 16)] = z16
        return 0
    lax.fori_loop(0, 64, zrow, 0)

    rows_per_tile = N_PAD // NS

    def zacc(k, _):
        pltpu.sync_copy(zbuf, acc.at[pl.ds(s * rows_per_tile + k * 64, 64)])
        return 0
    lax.fori_loop(0, rows_per_tile // 64, zacc, 0)

    def zpad(e, _):
        rowbuf[e, pl.ds(128, 16)] = z16
        return 0
    lax.fori_loop(0, CHUNK, zpad, 0)

    plsc.subcore_barrier()

    tile = c * NS + s
    lane = lax.iota(jnp.int32, 16)

    def chunk_body(k, _):
        base = (tile * n_chunks_per_tile + k) * CHUNK
        pltpu.sync_copy(src_hbm.at[pl.ds(base, CHUNK)], sidx)
        pltpu.sync_copy(dst_hbm.at[pl.ds(base, CHUNK)], didx)
        pltpu.async_copy(t1_hbm.at[sidx], hbuf, sem).wait()
        pltpu.async_copy(t2_hbm.at[didx], adbuf, sem).wait()
        for g in range(CHUNK // 16):
            eidx = lane + (g * 16)
            for hh in range(H):
                col = jnp.full((16,), 128 + hh, jnp.int32)
                a_s = plsc.load_gather(hbuf, [eidx, col])
                a_d = plsc.load_gather(adbuf, [eidx, jnp.full((16,), hh, jnp.int32)])
                e = a_s + a_d
                e = jnp.where(e >= 0.0, e, 0.2 * e)
                w = jnp.exp(e)
                plsc.store_scatter(rowbuf, [eidx, col], w)
        def edge_body(e, _):
            ws = [rowbuf[e, 128 + hh] for hh in range(H)]
            for j in range(8):
                rowbuf[e, pl.ds(16 * j, 16)] = hbuf[e, pl.ds(16 * j, 16)] * ws[j // 2]
            return 0
        lax.fori_loop(0, CHUNK, edge_body, 0)
        pltpu.sync_copy(rowbuf, acc.at[didx], add=True)
        return 0

    lax.fori_loop(0, n_chunks_per_tile, chunk_body, 0)

    plsc.subcore_barrier()
    pltpu.sync_copy(acc.at[pl.ds(s * rows_per_tile, rows_per_tile)],
                    out_hbm.at[c, pl.ds(s * rows_per_tile, rows_per_tile)])


def _k3_body(x_ref, a0_ref, a1_ref, r_ref, bg_ref, g1_ref, be1_ref,
             w1_ref, b1_ref, w2_ref, b2_ref, g2_ref, be2_ref, o_ref):
    S = a0_ref[...] + a1_ref[...]
    gat = S[:, :128]
    den = S[:, 128:132]
    den128 = jnp.dot(den, r_ref[...], preferred_element_type=jnp.float32,
                     precision=lax.Precision.HIGHEST) + 1e-16
    gat = gat / den128 + bg_ref[...]
    y = x_ref[...] + gat
    mu = jnp.mean(y, axis=-1, keepdims=True)
    var = jnp.mean((y - mu) ** 2, axis=-1, keepdims=True)
    y = (y - mu) / jnp.sqrt(var + 1e-5) * g1_ref[...] + be1_ref[...]
    hmid = jnp.dot(y, w1_ref[...], preferred_element_type=jnp.float32,
                   precision=lax.Precision.HIGHEST) + b1_ref[...]
    hmid = 0.5 * hmid * (1.0 + lax.erf(hmid * 0.7071067811865476))
    mlp = jnp.dot(hmid, w2_ref[...], preferred_element_type=jnp.float32,
                  precision=lax.Precision.HIGHEST) + b2_ref[...]
    z = y + mlp
    mu = jnp.mean(z, axis=-1, keepdims=True)
    var = jnp.mean((z - mu) ** 2, axis=-1, keepdims=True)
    o_ref[...] = (z - mu) / jnp.sqrt(var + 1e-5) * g2_ref[...] + be2_ref[...]


@jax.jit
def kernel(x, edge_index, W, att_src, att_dst, bias_gat, gamma1, beta1,
           W1, b1, W2, b2, gamma2, beta2):
    f32 = jnp.float32
    x_pad = jnp.pad(x, ((0, N_PAD - N), (0, 0)))
    att_s = att_src.reshape(-1)
    att_d = att_dst.reshape(-1)
    mask = (jnp.arange(D)[:, None] // C == jnp.arange(H)[None, :]).astype(f32)
    As = att_s[:, None] * mask
    Ad = att_d[:, None] * mask
    R = mask.T

    E = edge_index.shape[1]
    e_tot = E + N
    n_chunks_per_tile = -(-e_tot // (NW * CHUNK))
    e_pad = n_chunks_per_tile * NW * CHUNK
    loops = jnp.arange(N, dtype=jnp.int32)
    padv = jnp.full((e_pad - e_tot,), N, jnp.int32)
    src_ext = jnp.concatenate([edge_index[0], loops, padv])
    dst_ext = jnp.concatenate([edge_index[1], loops, padv])

    t1, t2 = pl.pallas_call(
        _k1_body,
        grid=(N_PAD // BLK,),
        in_specs=[
            pl.BlockSpec((BLK, D), lambda i: (i, 0)),
            pl.BlockSpec((D, D), lambda i: (0, 0)),
            pl.BlockSpec((D, H), lambda i: (0, 0)),
            pl.BlockSpec((D, H), lambda i: (0, 0)),
        ],
        out_specs=[
            pl.BlockSpec((BLK, ROWW), lambda i: (i, 0)),
            pl.BlockSpec((BLK, 16), lambda i: (i, 0)),
        ],
        out_shape=[
            jax.ShapeDtypeStruct((N_PAD, ROWW), f32),
            jax.ShapeDtypeStruct((N_PAD, 16), f32),
        ],
    )(x_pad, W, As, Ad)

    k2 = functools.partial(
        pl.kernel,
        out_type=jax.ShapeDtypeStruct((NC, N_PAD, ROWW), f32),
        mesh=plsc.VectorSubcoreMesh(core_axis_name="c", subcore_axis_name="s"),
        scratch_types=[
            pltpu.VMEM_SHARED((N_PAD, ROWW), f32),
            pltpu.VMEM((CHUNK, ROWW), f32),
            pltpu.VMEM((CHUNK, 16), f32),
            pltpu.VMEM((CHUNK, ROWW), f32),
            pltpu.VMEM((64, ROWW), f32),
            pltpu.VMEM((CHUNK,), jnp.int32),
            pltpu.VMEM((CHUNK,), jnp.int32),
            pltpu.SemaphoreType.DMA,
        ],
    )(functools.partial(_k2_body, n_chunks_per_tile=n_chunks_per_tile))
    parts = k2(t1, t2, src_ext, dst_ext)

    out = pl.pallas_call(
        _k3_body,
        grid=(N_PAD // BLK,),
        in_specs=[
            pl.BlockSpec((BLK, D), lambda i: (i, 0)),
            pl.BlockSpec((BLK, ROWW), lambda i: (i, 0)),
            pl.BlockSpec((BLK, ROWW), lambda i: (i, 0)),
            pl.BlockSpec((H, D), lambda i: (0, 0)),
            pl.BlockSpec((D,), lambda i: (0,)),
            pl.BlockSpec((D,), lambda i: (0,)),
            pl.BlockSpec((D,), lambda i: (0,)),
            pl.BlockSpec((D, MLPD), lambda i: (0, 0)),
            pl.BlockSpec((MLPD,), lambda i: (0,)),
            pl.BlockSpec((MLPD, D), lambda i: (0, 0)),
            pl.BlockSpec((D,), lambda i: (0,)),
            pl.BlockSpec((D,), lambda i: (0,)),
            pl.BlockSpec((D,), lambda i: (0,)),
        ],
        out_specs=pl.BlockSpec((BLK, D), lambda i: (i, 0)),
        out_shape=jax.ShapeDtypeStruct((N_PAD, D), f32),
    )(x_pad, parts[0], parts[1], R, bias_gat, gamma1, beta1,
      W1, b1, W2, b2, gamma2, beta2)

    return out[:N]


# speedup vs baseline: 62.2449x; 62.2449x over previous
import functools

import jax
import jax.numpy as jnp
from jax import lax
from jax.experimental import pallas as pl
from jax.experimental.pallas import tpu as pltpu
from jax.experimental.pallas import tpu_sc as plsc

N = 10000
D = 128
H = 4
C = 32
MLPD = 512

NC = 2
NS = 16
NW = NC * NS

N_PAD = 10240
ROWW = 144
CHUNK = 128
BLK = 1024

_NEG = -1e30


def _k1_body(x_ref, w_ref, as_ref, ad_ref, t1_ref, t2_ref):
    i = pl.program_id(0)
    h = jnp.dot(x_ref[...], w_ref[...], preferred_element_type=jnp.float32,
                precision=lax.Precision.HIGHEST)
    a_s = jnp.dot(h, as_ref[...], preferred_element_type=jnp.float32,
                  precision=lax.Precision.HIGHEST)
    a_d = jnp.dot(h, ad_ref[...], preferred_element_type=jnp.float32,
                  precision=lax.Precision.HIGHEST)
    rows = i * BLK + lax.broadcasted_iota(jnp.int32, (BLK, 1), 0)
    sent = rows >= N
    a_s = jnp.where(sent, _NEG, a_s)
    a_d = jnp.where(sent, _NEG, a_d)
    z12 = jnp.zeros((BLK, 12), jnp.float32)
    t1_ref[...] = jnp.concatenate([h, a_s, z12], axis=1)
    t2_ref[...] = jnp.concatenate([a_d, z12], axis=1)


def _k2_body(t1_hbm, t2_hbm, src_hbm, dst_hbm, out_hbm,
             acc, hbuf, adbuf, sidx, didx, sem,
             *, n_chunks_per_tile):
    c = lax.axis_index("c")
    s = lax.axis_index("s")
    z16 = jnp.zeros((16,), jnp.float32)

    def zrow(r, _):
        for j in range(ROWW // 16):
            hbuf[r, pl.ds(16 * j, 16)] = z16
        return 0
    lax.fori_loop(0, CHUNK, zrow, 0)

    rows_per_tile = N_PAD // NS

    def zacc(k, _):
        pltpu.sync_copy(hbuf.at[pl.ds(0, 64)],
                        acc.at[pl.ds(s * rows_per_tile + k * 64, 64)])
        return 0
    lax.fori_loop(0, rows_per_tile // 64, zacc, 0)

    plsc.subcore_barrier()

    tile = c * NS + s
    lane = lax.iota(jnp.int32, 16)

    def chunk_body(k, _):
        base = (tile * n_chunks_per_tile + k) * CHUNK
        pltpu.sync_copy(src_hbm.at[pl.ds(base, CHUNK)], sidx)
        pltpu.sync_copy(dst_hbm.at[pl.ds(base, CHUNK)], didx)
        pltpu.async_copy(t1_hbm.at[sidx], hbuf, sem).wait()
        pltpu.async_copy(t2_hbm.at[didx], adbuf, sem).wait()
        for g in range(CHUNK // 16):
            eidx = lane + (g * 16)
            for hh in range(H):
                col = jnp.full((16,), 128 + hh, jnp.int32)
                a_s = plsc.load_gather(hbuf, [eidx, col])
                a_d = plsc.load_gather(adbuf, [eidx, jnp.full((16,), hh, jnp.int32)])
                e = a_s + a_d
                e = jnp.where(e >= 0.0, e, 0.2 * e)
                w = jnp.exp(e)
                plsc.store_scatter(hbuf, [eidx, col], w)
        def edge_body(e, _):
            wv = hbuf[e, pl.ds(128, 16)]
            for j in range(8):
                hbuf[e, pl.ds(16 * j, 16)] = hbuf[e, pl.ds(16 * j, 16)] * wv[j // 2]
            return 0
        lax.fori_loop(0, CHUNK, edge_body, 0)
        pltpu.sync_copy(hbuf, acc.at[didx], add=True)
        return 0

    lax.fori_loop(0, n_chunks_per_tile, chunk_body, 0)

    plsc.subcore_barrier()
    pltpu.sync_copy(acc.at[pl.ds(s * rows_per_tile, rows_per_tile)],
                    out_hbm.at[c, pl.ds(s * rows_per_tile, rows_per_tile)])


def _k3_body(x_ref, a0_ref, a1_ref, r_ref, bg_ref, g1_ref, be1_ref,
             w1_ref, b1_ref, w2_ref, b2_ref, g2_ref, be2_ref, o_ref):
    S = a0_ref[...] + a1_ref[...]
    gat = S[:, :128]
    den = S[:, 128:132]
    den128 = jnp.dot(den, r_ref[...], preferred_element_type=jnp.float32,
                     precision=lax.Precision.HIGHEST) + 1e-16
    gat = gat / den128 + bg_ref[...]
    y = x_ref[...] + gat
    mu = jnp.mean(y, axis=-1, keepdims=True)
    var = jnp.mean((y - mu) ** 2, axis=-1, keepdims=True)
    y = (y - mu) / jnp.sqrt(var + 1e-5) * g1_ref[...] + be1_ref[...]
    hmid = jnp.dot(y, w1_ref[...], preferred_element_type=jnp.float32,
                   precision=lax.Precision.HIGHEST) + b1_ref[...]
    hmid = 0.5 * hmid * (1.0 + lax.erf(hmid * 0.7071067811865476))
    mlp = jnp.dot(hmid, w2_ref[...], preferred_element_type=jnp.float32,
                  precision=lax.Precision.HIGHEST) + b2_ref[...]
    z = y + mlp
    mu = jnp.mean(z, axis=-1, keepdims=True)
    var = jnp.mean((z - mu) ** 2, axis=-1, keepdims=True)
    o_ref[...] = (z - mu) / jnp.sqrt(var + 1e-5) * g2_ref[...] + be2_ref[...]


@jax.jit
def kernel(x, edge_index, W, att_src, att_dst, bias_gat, gamma1, beta1,
           W1, b1, W2, b2, gamma2, beta2):
    f32 = jnp.float32
    x_pad = jnp.pad(x, ((0, N_PAD - N), (0, 0)))
    att_s = att_src.reshape(-1)
    att_d = att_dst.reshape(-1)
    mask = (jnp.arange(D)[:, None] // C == jnp.arange(H)[None, :]).astype(f32)
    As = att_s[:, None] * mask
    Ad = att_d[:, None] * mask
    R = mask.T

    E = edge_index.shape[1]
    e_tot = E + N
    n_chunks_per_tile = -(-e_tot // (NW * CHUNK))
    e_pad = n_chunks_per_tile * NW * CHUNK
    loops = jnp.arange(N, dtype=jnp.int32)
    padv = jnp.full((e_pad - e_tot,), N, jnp.int32)
    src_ext = jnp.concatenate([edge_index[0], loops, padv])
    dst_ext = jnp.concatenate([edge_index[1], loops, padv])

    t1, t2 = pl.pallas_call(
        _k1_body,
        grid=(N_PAD // BLK,),
        in_specs=[
            pl.BlockSpec((BLK, D), lambda i: (i, 0)),
            pl.BlockSpec((D, D), lambda i: (0, 0)),
            pl.BlockSpec((D, H), lambda i: (0, 0)),
            pl.BlockSpec((D, H), lambda i: (0, 0)),
        ],
        out_specs=[
            pl.BlockSpec((BLK, ROWW), lambda i: (i, 0)),
            pl.BlockSpec((BLK, 16), lambda i: (i, 0)),
        ],
        out_shape=[
            jax.ShapeDtypeStruct((N_PAD, ROWW), f32),
            jax.ShapeDtypeStruct((N_PAD, 16), f32),
        ],
    )(x_pad, W, As, Ad)

    k2 = functools.partial(
        pl.kernel,
        out_type=jax.ShapeDtypeStruct((NC, N_PAD, ROWW), f32),
        mesh=plsc.VectorSubcoreMesh(core_axis_name="c", subcore_axis_name="s"),
        compiler_params=pltpu.CompilerParams(use_tc_tiling_on_sc=False,
                                             needs_layout_passes=False),
        scratch_types=[
            pltpu.VMEM_SHARED((N_PAD, ROWW), f32),
            pltpu.VMEM((CHUNK, ROWW), f32),
            pltpu.VMEM((CHUNK, 16), f32),
            pltpu.VMEM((CHUNK,), jnp.int32),
            pltpu.VMEM((CHUNK,), jnp.int32),
            pltpu.SemaphoreType.DMA,
        ],
    )(functools.partial(_k2_body, n_chunks_per_tile=n_chunks_per_tile))
    parts = k2(t1, t2, src_ext, dst_ext)

    out = pl.pallas_call(
        _k3_body,
        grid=(N_PAD // BLK,),
        in_specs=[
            pl.BlockSpec((BLK, D), lambda i: (i, 0)),
            pl.BlockSpec((BLK, ROWW), lambda i: (i, 0)),
            pl.BlockSpec((BLK, ROWW), lambda i: (i, 0)),
            pl.BlockSpec((H, D), lambda i: (0, 0)),
            pl.BlockSpec((D,), lambda i: (0,)),
            pl.BlockSpec((D,), lambda i: (0,)),
            pl.BlockSpec((D,), lambda i: (0,)),
            pl.BlockSpec((D, MLPD), lambda i: (0, 0)),
            pl.BlockSpec((MLPD,), lambda i: (0,)),
            pl.BlockSpec((MLPD, D), lambda i: (0, 0)),
            pl.BlockSpec((D,), lambda i: (0,)),
            pl.BlockSpec((D,), lambda i: (0,)),
            pl.BlockSpec((D,), lambda i: (0,)),
        ],
        out_specs=pl.BlockSpec((BLK, D), lambda i: (i, 0)),
        out_shape=jax.ShapeDtypeStruct((N_PAD, D), f32),
    )(x_pad, parts[0], parts[1], R, bias_gat, gamma1, beta1,
      W1, b1, W2, b2, gamma2, beta2)

    return out[:N]
